# Initial kernel scaffold; baseline (speedup 1.0000x reference)
import jax
import jax.numpy as jnp
from jax.experimental import pallas as pl


def kernel(x_flat, w_gate, w_noise):
    raise NotImplementedError("write your pallas kernel here")


# speedup vs baseline: 5.7676x; 5.7676x over previous
import functools

import jax
import jax.numpy as jnp
from jax.experimental import pallas as pl

IN_DIM = 768
NUM_EXPERTS = 64
NUM_TOKENS = 32768
BLOCK_ROWS = 2048


def _router_body(x_ref, w_ref, logits_ref, gates_ref, imp_ref, load_ref,
                 z_ref, lb_ref):
    i = pl.program_id(0)
    logits = jnp.dot(x_ref[...], w_ref[...], preferred_element_type=jnp.float32)
    logits_ref[...] = logits

    iota = jax.lax.broadcasted_iota(jnp.int32, logits.shape, 1)
    m1 = jnp.max(logits, axis=1, keepdims=True)
    i1 = jnp.min(jnp.where(logits == m1, iota, NUM_EXPERTS), axis=1,
                 keepdims=True)
    masked = jnp.where(iota == i1, -jnp.inf, logits)
    m2 = jnp.max(masked, axis=1, keepdims=True)
    i2 = jnp.min(jnp.where(masked == m2, iota, NUM_EXPERTS), axis=1,
                 keepdims=True)

    t = jnp.exp(m2 - m1)
    denom = 1.0 + t
    g1 = 1.0 / denom
    g2 = t / denom
    gates = (jnp.where(iota == i1, g1, 0.0) + jnp.where(iota == i2, g2, 0.0))
    gates_ref[...] = gates

    zpart = jnp.sum(jnp.log(jnp.sum(jnp.exp(logits), axis=1)))
    imp_part = jnp.sum(gates, axis=0, keepdims=True)
    load_part = jnp.sum((gates > 0).astype(jnp.int32), axis=0, keepdims=True)

    @pl.when(i == 0)
    def _init():
        imp_ref[...] = jnp.zeros_like(imp_ref)
        load_ref[...] = jnp.zeros_like(load_ref)
        z_ref[...] = jnp.zeros_like(z_ref)

    imp_ref[...] += imp_part
    load_ref[...] += load_part
    z_ref[...] += zpart

    @pl.when(i == pl.num_programs(0) - 1)
    def _finalize():
        def cv_sq(v):
            mean = jnp.mean(v)
            var = jnp.sum((v - mean) ** 2) / (v.size - 1)
            return var / (mean * mean + 1e-10)

        imp = imp_ref[0, :]
        loadf = load_ref[0, :].astype(jnp.float32)
        zl = z_ref[0, 0] / NUM_TOKENS
        lb_ref[...] = (cv_sq(imp) + cv_sq(loadf) + zl).reshape(1, 1)


@functools.partial(jax.jit, static_argnames=())
def _router(x_flat, w_gate):
    grid = NUM_TOKENS // BLOCK_ROWS
    out = pl.pallas_call(
        _router_body,
        grid=(grid,),
        in_specs=[
            pl.BlockSpec((BLOCK_ROWS, IN_DIM), lambda i: (i, 0)),
            pl.BlockSpec((IN_DIM, NUM_EXPERTS), lambda i: (0, 0)),
        ],
        out_specs=[
            pl.BlockSpec((BLOCK_ROWS, NUM_EXPERTS), lambda i: (i, 0)),
            pl.BlockSpec((BLOCK_ROWS, NUM_EXPERTS), lambda i: (i, 0)),
            pl.BlockSpec((1, NUM_EXPERTS), lambda i: (0, 0)),
            pl.BlockSpec((1, NUM_EXPERTS), lambda i: (0, 0)),
            pl.BlockSpec((1, 1), lambda i: (0, 0)),
            pl.BlockSpec((1, 1), lambda i: (0, 0)),
        ],
        out_shape=[
            jax.ShapeDtypeStruct((NUM_TOKENS, NUM_EXPERTS), jnp.float32),
            jax.ShapeDtypeStruct((NUM_TOKENS, NUM_EXPERTS), jnp.float32),
            jax.ShapeDtypeStruct((1, NUM_EXPERTS), jnp.float32),
            jax.ShapeDtypeStruct((1, NUM_EXPERTS), jnp.int32),
            jax.ShapeDtypeStruct((1, 1), jnp.float32),
            jax.ShapeDtypeStruct((1, 1), jnp.float32),
        ],
    )(x_flat, w_gate)
    return out


def kernel(x_flat, w_gate, w_noise):
    del w_noise
    logits, gates, imp, load, _z, lb = _router(x_flat, w_gate)
    return (gates, load.reshape(NUM_EXPERTS), logits, lb.reshape(()),
            imp.reshape(NUM_EXPERTS))


# speedup vs baseline: 6.1074x; 1.0589x over previous
import functools

import jax
import jax.numpy as jnp
from jax.experimental import pallas as pl

IN_DIM = 768
NUM_EXPERTS = 64
NUM_TOKENS = 32768
BLOCK_ROWS = 4096


def _router_body(x_ref, w_ref, logits_ref, gates_ref, imp_ref, load_ref,
                 z_ref, lb_ref):
    i = pl.program_id(0)
    logits = jnp.dot(x_ref[...], w_ref[...], preferred_element_type=jnp.float32)
    logits_ref[...] = logits

    iota = jax.lax.broadcasted_iota(jnp.int32, logits.shape, 1)
    m1 = jnp.max(logits, axis=1, keepdims=True)
    i1 = jnp.min(jnp.where(logits == m1, iota, NUM_EXPERTS), axis=1,
                 keepdims=True)
    masked = jnp.where(iota == i1, -jnp.inf, logits)
    m2 = jnp.max(masked, axis=1, keepdims=True)
    i2 = jnp.min(jnp.where(masked == m2, iota, NUM_EXPERTS), axis=1,
                 keepdims=True)

    t = jnp.exp(m2 - m1)
    denom = 1.0 + t
    g1 = 1.0 / denom
    g2 = t / denom
    gates = (jnp.where(iota == i1, g1, 0.0) + jnp.where(iota == i2, g2, 0.0))
    gates_ref[...] = gates

    zpart = jnp.sum(jnp.log(jnp.sum(jnp.exp(logits), axis=1)))
    imp_part = jnp.sum(gates, axis=0, keepdims=True)
    load_part = jnp.sum((gates > 0).astype(jnp.int32), axis=0, keepdims=True)

    @pl.when(i == 0)
    def _init():
        imp_ref[...] = jnp.zeros_like(imp_ref)
        load_ref[...] = jnp.zeros_like(load_ref)
        z_ref[...] = jnp.zeros_like(z_ref)

    imp_ref[...] += imp_part
    load_ref[...] += load_part
    z_ref[...] += zpart

    @pl.when(i == pl.num_programs(0) - 1)
    def _finalize():
        def cv_sq(v):
            mean = jnp.mean(v)
            var = jnp.sum((v - mean) ** 2) / (v.size - 1)
            return var / (mean * mean + 1e-10)

        imp = imp_ref[0, :]
        loadf = load_ref[0, :].astype(jnp.float32)
        zl = z_ref[0, 0] / NUM_TOKENS
        lb_ref[...] = (cv_sq(imp) + cv_sq(loadf) + zl).reshape(1, 1)


@functools.partial(jax.jit, static_argnames=())
def _router(x_flat, w_gate):
    grid = NUM_TOKENS // BLOCK_ROWS
    out = pl.pallas_call(
        _router_body,
        grid=(grid,),
        in_specs=[
            pl.BlockSpec((BLOCK_ROWS, IN_DIM), lambda i: (i, 0)),
            pl.BlockSpec((IN_DIM, NUM_EXPERTS), lambda i: (0, 0)),
        ],
        out_specs=[
            pl.BlockSpec((BLOCK_ROWS, NUM_EXPERTS), lambda i: (i, 0)),
            pl.BlockSpec((BLOCK_ROWS, NUM_EXPERTS), lambda i: (i, 0)),
            pl.BlockSpec((1, NUM_EXPERTS), lambda i: (0, 0)),
            pl.BlockSpec((1, NUM_EXPERTS), lambda i: (0, 0)),
            pl.BlockSpec((1, 1), lambda i: (0, 0)),
            pl.BlockSpec((1, 1), lambda i: (0, 0)),
        ],
        out_shape=[
            jax.ShapeDtypeStruct((NUM_TOKENS, NUM_EXPERTS), jnp.float32),
            jax.ShapeDtypeStruct((NUM_TOKENS, NUM_EXPERTS), jnp.float32),
            jax.ShapeDtypeStruct((1, NUM_EXPERTS), jnp.float32),
            jax.ShapeDtypeStruct((1, NUM_EXPERTS), jnp.int32),
            jax.ShapeDtypeStruct((1, 1), jnp.float32),
            jax.ShapeDtypeStruct((1, 1), jnp.float32),
        ],
    )(x_flat, w_gate)
    return out


def kernel(x_flat, w_gate, w_noise):
    del w_noise
    logits, gates, imp, load, _z, lb = _router(x_flat, w_gate)
    return (gates, load.reshape(NUM_EXPERTS), logits, lb.reshape(()),
            imp.reshape(NUM_EXPERTS))
